# baseline (speedup 1.0000x reference)
import functools

import jax
import jax.numpy as jnp
from jax import lax
from jax.experimental import pallas as pl
from jax.experimental.pallas import tpu as pltpu
from jax.experimental.pallas import tpu_sc as plsc

_NUM_DENSE = 1024
_IFEAT = 64
_ICTX = 16
_HID = 128
_SPD = 32

_NC = 2
_NS = 16
_NW = _NC * _NS
_LANES = 16
_CH = 128


def _sc_gather_call(tokens3, if2, sp4, dense_table, n, spread):
    pw = n // _NW
    nch = pw // _CH
    ng = nch // 2
    dn_rows_tot = dense_table.shape[0]

    mesh = plsc.VectorSubcoreMesh(core_axis_name="c", subcore_axis_name="s")

    @functools.partial(
        pl.kernel,
        mesh=mesh,
        compiler_params=pltpu.CompilerParams(use_tc_tiling_on_sc=True),
        out_type=[
            jax.ShapeDtypeStruct((n, _HID), jnp.float32),
            jax.ShapeDtypeStruct((n, _HID), jnp.float32),
            jax.ShapeDtypeStruct((n, _HID), jnp.float32),
            jax.ShapeDtypeStruct((_NW, nch, _CH), jnp.float32),
        ],
        scratch_types=[
            pltpu.VMEM((nch, _CH), jnp.int32),
            pltpu.VMEM((nch, _CH), jnp.int32),
            pltpu.VMEM((nch, _CH), jnp.int32),
            pltpu.VMEM((nch, _CH), jnp.float32),
            pltpu.VMEM((_CH, _HID), jnp.float32),
            pltpu.VMEM((_CH, _HID), jnp.float32),
            pltpu.VMEM((_CH, _HID), jnp.float32),
            pltpu.VMEM((_CH, _HID), jnp.float32),
            pltpu.VMEM((_CH, _HID), jnp.float32),
            pltpu.VMEM((_CH, _HID), jnp.float32),
            pltpu.SemaphoreType.DMA,
            pltpu.SemaphoreType.DMA,
            pltpu.SemaphoreType.DMA,
            pltpu.SemaphoreType.DMA,
            pltpu.SemaphoreType.DMA,
        ],
    )
    def sc_gather(tok_hbm, if_hbm, sp_hbm, dn_hbm,
                  if_out, sp_out, dn_out, ck_out,
                  qall, ifiall, dniall, code,
                  ifr0, spr0, dnr0, ifr1, spr1, dnr1,
                  gsem0, gsem1, ssem0, ssem1, msem):
        wid = lax.axis_index("s") * _NC + lax.axis_index("c")
        base = wid * pw

        pltpu.sync_copy(tok_hbm.at[wid], qall)

        def idx_chunk(c, carry):
            for i in range(_CH // _LANES):
                sl = pl.ds(i * _LANES, _LANES)
                t = qall[c, sl]
                is_sp = t > _NUM_DENSE
                spi = jnp.where(is_sp, t - _NUM_DENSE, t + spread)
                m = jnp.where(is_sp, 1, 0)
                par = t & 1
                gid = spi & 3
                code[c, sl] = (m + 2 * par + 8 * gid).astype(jnp.float32)
                dniall[c, sl] = lax.rem(t, dn_rows_tot)
                ifiall[c, sl] = t >> 1
                qall[c, sl] = spi >> 2
            return carry

        lax.fori_loop(0, nch, idx_chunk, 0)

        pltpu.async_copy(code, ck_out.at[wid], msem)

        def fire(c, ifr, spr, dnr, gsem):
            pltpu.async_copy(if_hbm.at[ifiall.at[c]], ifr, gsem)
            pltpu.async_copy(sp_hbm.at[qall.at[c]], spr, gsem)
            pltpu.async_copy(dn_hbm.at[dniall.at[c]], dnr, gsem)

        def wait_gathers(c, ifr, spr, dnr, gsem):
            pltpu.make_async_copy(if_hbm.at[ifiall.at[c]], ifr, gsem).wait()
            pltpu.make_async_copy(sp_hbm.at[qall.at[c]], spr, gsem).wait()
            pltpu.make_async_copy(dn_hbm.at[dniall.at[c]], dnr, gsem).wait()

        def fire_scatters(c, ifr, spr, dnr, ssem):
            off = base + c * _CH
            pltpu.async_copy(ifr, if_out.at[pl.ds(off, _CH)], ssem)
            pltpu.async_copy(spr, sp_out.at[pl.ds(off, _CH)], ssem)
            pltpu.async_copy(dnr, dn_out.at[pl.ds(off, _CH)], ssem)

        def wait_scatters(c, ifr, spr, dnr, ssem):
            off = base + c * _CH
            pltpu.make_async_copy(ifr, if_out.at[pl.ds(off, _CH)], ssem).wait()
            pltpu.make_async_copy(spr, sp_out.at[pl.ds(off, _CH)], ssem).wait()
            pltpu.make_async_copy(dnr, dn_out.at[pl.ds(off, _CH)], ssem).wait()

        fire(0, ifr0, spr0, dnr0, gsem0)
        fire(1, ifr1, spr1, dnr1, gsem1)

        def group(g, carry):
            c0 = 2 * g
            c1 = c0 + 1
            wait_gathers(c0, ifr0, spr0, dnr0, gsem0)
            fire_scatters(c0, ifr0, spr0, dnr0, ssem0)
            wait_gathers(c1, ifr1, spr1, dnr1, gsem1)
            fire_scatters(c1, ifr1, spr1, dnr1, ssem1)

            @pl.when(g + 1 < ng)
            def _():
                wait_scatters(c0, ifr0, spr0, dnr0, ssem0)
                fire(c0 + 2, ifr0, spr0, dnr0, gsem0)
                wait_scatters(c1, ifr1, spr1, dnr1, ssem1)
                fire(c1 + 2, ifr1, spr1, dnr1, gsem1)

            return carry

        lax.fori_loop(0, ng, group, 0)

        wait_scatters(nch - 2, ifr0, spr0, dnr0, ssem0)
        wait_scatters(nch - 1, ifr1, spr1, dnr1, ssem1)
        pltpu.make_async_copy(code, ck_out.at[wid], msem).wait()

    return sc_gather(tokens3, if2, sp4, dense_table)


def _tc_body(if_ref, ic_ref, sp_ref, dn_ref, ck_ref,
             wca2_ref, wcb_ref, wup4_ref, w1_ref, be_ref, out_ref):
    dot = functools.partial(
        jnp.dot, preferred_element_type=jnp.float32,
        precision=jax.lax.Precision.HIGHEST)
    codev = ck_ref[...]
    gid = jnp.floor(codev * 0.125)
    rem = codev - 8.0 * gid
    par = jnp.floor(rem * 0.5)
    m = rem - 2.0 * par
    lane = lax.broadcasted_iota(jnp.int32, (1, _HID), 1)
    half = (lane // 64).astype(jnp.float32)
    quad = (lane // 32).astype(jnp.float32)
    pmask = 1.0 - jnp.abs(par - half)
    gmask = jnp.maximum(0.0, 1.0 - jnp.abs(gid - quad))
    acc = dot(if_ref[...] * pmask, wca2_ref[...]) + dot(ic_ref[...],
                                                        wcb_ref[...])
    spv = dot(sp_ref[...] * gmask, wup4_ref[...])
    tv = jnp.where(m > 0.5, spv, dn_ref[...])
    v = acc + dot(tv, w1_ref[...]) + be_ref[...]
    s = jnp.sum(v * v, axis=1, keepdims=True)
    nrm = jnp.maximum(jnp.sqrt(s), 1e-12)
    out_ref[...] = v / nrm


def _tc_call(if_g, ic2, sp_g, dn_g, ck2, wca2, wcb, wup4, w1, beff, n):
    t = 512
    g = n // t
    const = lambda shape: pl.BlockSpec(shape, lambda i: (0, 0))
    row = lambda d: pl.BlockSpec((t, d), lambda i: (i, 0))
    return pl.pallas_call(
        _tc_body,
        grid=(g,),
        in_specs=[
            row(_HID), row(_ICTX), row(_HID), row(_HID), row(1),
            const((_HID, _HID)), const((_ICTX, _HID)),
            const((_HID, _HID)), const((_HID, _HID)), const((1, _HID)),
        ],
        out_specs=row(_HID),
        out_shape=jax.ShapeDtypeStruct((n, _HID), jnp.float32),
    )(if_g, ic2, sp_g, dn_g, ck2, wca2, wcb, wup4, w1, beff)


def kernel(tokens, icontexts, ifeatures, dense_table, sparse_table,
           W_up, W_ac, b_ac, W_item, b_item):
    b, l = tokens.shape
    n = b * l
    pw = n // _NW
    nch = pw // _CH
    sparse_rows = sparse_table.shape[0]
    sp_rows_pad = -(-sparse_rows // 4) * 4
    spread = sp_rows_pad - 1 - _NUM_DENSE

    w1 = W_item[:_HID]
    w2 = W_item[_HID:]
    wc = W_ac @ w2
    beff = (b_item + b_ac @ w2).reshape(1, _HID)
    wca2 = jnp.concatenate([wc[:_IFEAT], wc[:_IFEAT]], axis=0)
    wcb = wc[_IFEAT:]
    wup4 = jnp.concatenate([W_up, W_up, W_up, W_up], axis=0)

    if2 = ifeatures.reshape(ifeatures.shape[0] // 2, 2 * _IFEAT)
    sp4 = jnp.pad(sparse_table, ((0, sp_rows_pad - sparse_rows), (0, 0))
                  ).reshape(sp_rows_pad // 4, 4 * _SPD)

    tokens3 = tokens.reshape(_NW, nch, _CH).astype(jnp.int32)
    if_g, sp_g, dn_g, ck = _sc_gather_call(
        tokens3, if2, sp4, dense_table, n, spread)
    out = _tc_call(if_g, icontexts.reshape(n, _ICTX), sp_g, dn_g,
                   ck.reshape(n, 1), wca2, wcb, wup4, w1, beff, n)
    return out.reshape(b, l, _HID)


# speedup vs baseline: 1.3437x; 1.3437x over previous
import functools

import jax
import jax.numpy as jnp
from jax import lax
from jax.experimental import pallas as pl
from jax.experimental.pallas import tpu as pltpu
from jax.experimental.pallas import tpu_sc as plsc

_NUM_DENSE = 1024
_IFEAT = 64
_ICTX = 16
_HID = 128
_SPD = 32

_NC = 2
_NS = 16
_NW = _NC * _NS
_LANES = 16
_CH = 128


def _sc_gather_call(tokens3, if_pad, sp_pad, n, spread):
    pw = n // _NW
    nch = pw // _CH
    ng = nch // 2

    mesh = plsc.VectorSubcoreMesh(core_axis_name="c", subcore_axis_name="s")

    @functools.partial(
        pl.kernel,
        mesh=mesh,
        compiler_params=pltpu.CompilerParams(use_tc_tiling_on_sc=True),
        out_type=[
            jax.ShapeDtypeStruct((n, _HID), jnp.float32),
            jax.ShapeDtypeStruct((n, _HID), jnp.float32),
        ],
        scratch_types=[
            pltpu.VMEM((nch, _CH), jnp.int32),
            pltpu.VMEM((nch, _CH), jnp.int32),
            pltpu.VMEM((_CH, _HID), jnp.float32),
            pltpu.VMEM((_CH, _HID), jnp.float32),
            pltpu.VMEM((_CH, _HID), jnp.float32),
            pltpu.VMEM((_CH, _HID), jnp.float32),
            pltpu.SemaphoreType.DMA,
            pltpu.SemaphoreType.DMA,
            pltpu.SemaphoreType.DMA,
            pltpu.SemaphoreType.DMA,
        ],
    )
    def sc_gather(tok_hbm, if_hbm, sp_hbm,
                  if_out, sp_out,
                  tokall, spiall,
                  ifr0, spr0, ifr1, spr1,
                  gsem0, gsem1, ssem0, ssem1):
        wid = lax.axis_index("s") * _NC + lax.axis_index("c")
        base = wid * pw

        pltpu.sync_copy(tok_hbm.at[wid], tokall)

        def idx_chunk(c, carry):
            for i in range(_CH // _LANES):
                sl = pl.ds(i * _LANES, _LANES)
                t = tokall[c, sl]
                spiall[c, sl] = jnp.where(t > _NUM_DENSE, t - _NUM_DENSE,
                                          t + spread)
            return carry

        lax.fori_loop(0, nch, idx_chunk, 0)

        def fire(c, ifr, spr, gsem):
            pltpu.async_copy(if_hbm.at[tokall.at[c]], ifr, gsem)
            pltpu.async_copy(sp_hbm.at[spiall.at[c]], spr, gsem)

        def wait_gathers(c, ifr, spr, gsem):
            pltpu.make_async_copy(if_hbm.at[tokall.at[c]], ifr, gsem).wait()
            pltpu.make_async_copy(sp_hbm.at[spiall.at[c]], spr, gsem).wait()

        def fire_scatters(c, ifr, spr, ssem):
            off = base + c * _CH
            pltpu.async_copy(ifr, if_out.at[pl.ds(off, _CH)], ssem)
            pltpu.async_copy(spr, sp_out.at[pl.ds(off, _CH)], ssem)

        def wait_scatters(c, ifr, spr, ssem):
            off = base + c * _CH
            pltpu.make_async_copy(ifr, if_out.at[pl.ds(off, _CH)], ssem).wait()
            pltpu.make_async_copy(spr, sp_out.at[pl.ds(off, _CH)], ssem).wait()

        fire(0, ifr0, spr0, gsem0)
        fire(1, ifr1, spr1, gsem1)

        def group(g, carry):
            c0 = 2 * g
            c1 = c0 + 1
            wait_gathers(c0, ifr0, spr0, gsem0)
            fire_scatters(c0, ifr0, spr0, ssem0)
            wait_gathers(c1, ifr1, spr1, gsem1)
            fire_scatters(c1, ifr1, spr1, ssem1)

            @pl.when(g + 1 < ng)
            def _():
                wait_scatters(c0, ifr0, spr0, ssem0)
                fire(c0 + 2, ifr0, spr0, gsem0)
                wait_scatters(c1, ifr1, spr1, ssem1)
                fire(c1 + 2, ifr1, spr1, gsem1)

            return carry

        lax.fori_loop(0, ng, group, 0)

        wait_scatters(nch - 2, ifr0, spr0, ssem0)
        wait_scatters(nch - 1, ifr1, spr1, ssem1)

    return sc_gather(tokens3, if_pad, sp_pad)


def _tc_body(if_ref, ic_ref, sp_ref, tk_ref,
             wca_ref, wcb_ref, wsp_ref, d1_ref, be_ref, out_ref):
    dot = functools.partial(
        jnp.dot, preferred_element_type=jnp.float32,
        precision=jax.lax.Precision.HIGHEST)
    tok = tk_ref[...]
    acc = dot(if_ref[...], wca_ref[...]) + dot(ic_ref[...], wcb_ref[...])
    spw = dot(sp_ref[...], wsp_ref[...])
    dni = lax.rem(tok, d1_ref.shape[0])
    rows = lax.broadcasted_iota(jnp.int32, (1, d1_ref.shape[0]), 1)
    onehot = (dni == rows).astype(jnp.float32)
    dnc = jnp.dot(onehot, d1_ref[...],
                  preferred_element_type=jnp.float32)
    tv = jnp.where(tok > _NUM_DENSE, spw, dnc)
    v = acc + tv + be_ref[...]
    s = jnp.sum(v * v, axis=1, keepdims=True)
    nrm = jnp.maximum(jnp.sqrt(s), 1e-12)
    out_ref[...] = v / nrm


def _tc_call(if_g, ic2, sp_g, tk2, wca, wcb, wsp, d1, beff, n):
    t = 512
    g = n // t
    rtab = d1.shape[0]
    const = lambda shape: pl.BlockSpec(shape, lambda i: (0, 0))
    row = lambda d: pl.BlockSpec((t, d), lambda i: (i, 0))
    return pl.pallas_call(
        _tc_body,
        grid=(g,),
        in_specs=[
            row(_HID), row(_ICTX), row(_HID), row(1),
            const((_HID, _HID)), const((_ICTX, _HID)),
            const((_HID, _HID)), const((rtab, _HID)), const((1, _HID)),
        ],
        out_specs=row(_HID),
        out_shape=jax.ShapeDtypeStruct((n, _HID), jnp.float32),
    )(if_g, ic2, sp_g, tk2, wca, wcb, wsp, d1, beff)


def kernel(tokens, icontexts, ifeatures, dense_table, sparse_table,
           W_up, W_ac, b_ac, W_item, b_item):
    b, l = tokens.shape
    n = b * l
    pw = n // _NW
    nch = pw // _CH
    sparse_rows = sparse_table.shape[0]
    spread = sparse_rows - 1 - _NUM_DENSE

    w1 = W_item[:_HID]
    w2 = W_item[_HID:]
    wc = W_ac @ w2
    beff = (b_item + b_ac @ w2).reshape(1, _HID)
    wca = jnp.pad(wc[:_IFEAT], ((0, _HID - _IFEAT), (0, 0)))
    wcb = wc[_IFEAT:]
    wsp = jnp.pad(W_up @ w1, ((0, _HID - _SPD), (0, 0)))
    d1 = dense_table @ w1

    if_pad = jnp.pad(ifeatures, ((0, 0), (0, _HID - _IFEAT)))
    sp_pad = jnp.pad(sparse_table, ((0, 0), (0, _HID - _SPD)))

    tokens_flat = tokens.reshape(n).astype(jnp.int32)
    tokens3 = tokens_flat.reshape(_NW, nch, _CH)
    if_g, sp_g = _sc_gather_call(tokens3, if_pad, sp_pad, n, spread)
    out = _tc_call(if_g, icontexts.reshape(n, _ICTX), sp_g,
                   tokens_flat.reshape(n, 1), wca, wcb, wsp, d1, beff, n)
    return out.reshape(b, l, _HID)


# speedup vs baseline: 1.4203x; 1.0570x over previous
import functools

import jax
import jax.numpy as jnp
from jax import lax
from jax.experimental import pallas as pl
from jax.experimental.pallas import tpu as pltpu
from jax.experimental.pallas import tpu_sc as plsc

_NUM_DENSE = 1024
_IFEAT = 64
_ICTX = 16
_HID = 128
_SPD = 32

_NC = 2
_NS = 16
_NW = _NC * _NS
_LANES = 16
_CH = 128


def _sc_gather_call(tokens3, if_pad, sp_pad, n, spread):
    pw = n // _NW
    nch = pw // _CH
    ng = nch // 2

    mesh = plsc.VectorSubcoreMesh(core_axis_name="c", subcore_axis_name="s")

    @functools.partial(
        pl.kernel,
        mesh=mesh,
        compiler_params=pltpu.CompilerParams(use_tc_tiling_on_sc=True),
        out_type=[
            jax.ShapeDtypeStruct((n, _HID), jnp.float32),
            jax.ShapeDtypeStruct((n, _HID), jnp.float32),
        ],
        scratch_types=[
            pltpu.VMEM((nch, _CH), jnp.int32),
            pltpu.VMEM((nch, _CH), jnp.int32),
            pltpu.VMEM((_CH, _HID), jnp.float32),
            pltpu.VMEM((_CH, _HID), jnp.float32),
            pltpu.VMEM((_CH, _HID), jnp.float32),
            pltpu.VMEM((_CH, _HID), jnp.float32),
            pltpu.SemaphoreType.DMA,
            pltpu.SemaphoreType.DMA,
            pltpu.SemaphoreType.DMA,
            pltpu.SemaphoreType.DMA,
        ],
    )
    def sc_gather(tok_hbm, if_hbm, sp_hbm,
                  if_out, sp_out,
                  tokall, spiall,
                  ifr0, spr0, ifr1, spr1,
                  gsem0, gsem1, ssem0, ssem1):
        wid = lax.axis_index("s") * _NC + lax.axis_index("c")
        base = wid * pw

        pltpu.sync_copy(tok_hbm.at[wid], tokall)

        def idx_chunk(c, carry):
            for i in range(_CH // _LANES):
                sl = pl.ds(i * _LANES, _LANES)
                t = tokall[c, sl]
                spiall[c, sl] = jnp.where(t > _NUM_DENSE, t - _NUM_DENSE,
                                          t + spread)
            return carry

        lax.fori_loop(0, nch, idx_chunk, 0)

        def fire(c, ifr, spr, gsem):
            pltpu.async_copy(if_hbm.at[tokall.at[c]], ifr, gsem)
            pltpu.async_copy(sp_hbm.at[spiall.at[c]], spr, gsem)

        def wait_gathers(c, ifr, spr, gsem):
            pltpu.make_async_copy(if_hbm.at[tokall.at[c]], ifr, gsem).wait()
            pltpu.make_async_copy(sp_hbm.at[spiall.at[c]], spr, gsem).wait()

        def fire_scatters(c, ifr, spr, ssem):
            off = base + c * _CH
            pltpu.async_copy(ifr, if_out.at[pl.ds(off, _CH)], ssem)
            pltpu.async_copy(spr, sp_out.at[pl.ds(off, _CH)], ssem)

        def wait_scatters(c, ifr, spr, ssem):
            off = base + c * _CH
            pltpu.make_async_copy(ifr, if_out.at[pl.ds(off, _CH)], ssem).wait()
            pltpu.make_async_copy(spr, sp_out.at[pl.ds(off, _CH)], ssem).wait()

        fire(0, ifr0, spr0, gsem0)
        fire(1, ifr1, spr1, gsem1)

        def group(g, carry):
            c0 = 2 * g
            c1 = c0 + 1
            wait_gathers(c0, ifr0, spr0, gsem0)
            fire_scatters(c0, ifr0, spr0, ssem0)
            wait_gathers(c1, ifr1, spr1, gsem1)
            fire_scatters(c1, ifr1, spr1, ssem1)

            @pl.when(g + 1 < ng)
            def _():
                wait_scatters(c0, ifr0, spr0, ssem0)
                fire(c0 + 2, ifr0, spr0, gsem0)
                wait_scatters(c1, ifr1, spr1, ssem1)
                fire(c1 + 2, ifr1, spr1, gsem1)

            return carry

        lax.fori_loop(0, ng, group, 0)

        wait_scatters(nch - 2, ifr0, spr0, ssem0)
        wait_scatters(nch - 1, ifr1, spr1, ssem1)

    return sc_gather(tokens3, if_pad, sp_pad)


def _tc_body(if_ref, ic_ref, sp_ref, tk_ref,
             wca_ref, wcb_ref, wsp_ref, d1_ref, be_ref, out_ref):
    dot = functools.partial(jnp.dot, preferred_element_type=jnp.float32)
    tok = tk_ref[...]
    acc = dot(if_ref[...], wca_ref[...]) + dot(ic_ref[...], wcb_ref[...])
    spw = dot(sp_ref[...], wsp_ref[...])
    dni = lax.rem(tok, d1_ref.shape[0])
    rows = lax.broadcasted_iota(jnp.int32, (1, d1_ref.shape[0]), 1)
    onehot = (dni == rows).astype(jnp.float32)
    dnc = jnp.dot(onehot, d1_ref[...],
                  preferred_element_type=jnp.float32)
    tv = jnp.where(tok > _NUM_DENSE, spw, dnc)
    v = acc + tv + be_ref[...]
    s = jnp.sum(v * v, axis=1, keepdims=True)
    nrm = jnp.maximum(jnp.sqrt(s), 1e-12)
    out_ref[...] = v / nrm


def _tc_call(if_g, ic2, sp_g, tk2, wca, wcb, wsp, d1, beff, n):
    t = 512
    g = n // t
    rtab = d1.shape[0]
    const = lambda shape: pl.BlockSpec(shape, lambda i: (0, 0))
    row = lambda d: pl.BlockSpec((t, d), lambda i: (i, 0))
    return pl.pallas_call(
        _tc_body,
        grid=(g,),
        in_specs=[
            row(_HID), row(_ICTX), row(_HID), row(1),
            const((_HID, _HID)), const((_ICTX, _HID)),
            const((_HID, _HID)), const((rtab, _HID)), const((1, _HID)),
        ],
        out_specs=row(_HID),
        out_shape=jax.ShapeDtypeStruct((n, _HID), jnp.float32),
    )(if_g, ic2, sp_g, tk2, wca, wcb, wsp, d1, beff)


def kernel(tokens, icontexts, ifeatures, dense_table, sparse_table,
           W_up, W_ac, b_ac, W_item, b_item):
    b, l = tokens.shape
    n = b * l
    pw = n // _NW
    nch = pw // _CH
    sparse_rows = sparse_table.shape[0]
    spread = sparse_rows - 1 - _NUM_DENSE

    w1 = W_item[:_HID]
    w2 = W_item[_HID:]
    wc = W_ac @ w2
    beff = (b_item + b_ac @ w2).reshape(1, _HID)
    wca = jnp.pad(wc[:_IFEAT], ((0, _HID - _IFEAT), (0, 0)))
    wcb = wc[_IFEAT:]
    wsp = jnp.pad(W_up @ w1, ((0, _HID - _SPD), (0, 0)))
    d1 = dense_table @ w1

    if_pad = jnp.pad(ifeatures, ((0, 0), (0, _HID - _IFEAT)))
    sp_pad = jnp.pad(sparse_table, ((0, 0), (0, _HID - _SPD)))

    tokens_flat = tokens.reshape(n).astype(jnp.int32)
    tokens3 = tokens_flat.reshape(_NW, nch, _CH)
    if_g, sp_g = _sc_gather_call(tokens3, if_pad, sp_pad, n, spread)
    out = _tc_call(if_g, icontexts.reshape(n, _ICTX), sp_g,
                   tokens_flat.reshape(n, 1), wca, wcb, wsp, d1, beff, n)
    return out.reshape(b, l, _HID)
